# Initial kernel scaffold; baseline (speedup 1.0000x reference)
import jax
import jax.numpy as jnp
from jax.experimental import pallas as pl


def kernel(p_up_W1, p_up_b1, p_up_W2, p_up_b2, p_dn_W1, p_dn_b1, p_dn_W2, p_dn_b2, p_cu_W1, p_cu_b1, p_cu_W2, p_cu_b2, p_cd_W1, p_cd_b1, p_cd_W2, p_cd_b2, p_cell_W1, p_cell_b1, p_cell_W2, p_cell_b2, p_cw, h, h_up, h_down, x, x_up, x_down, b_up_i, b_up_j, b_down_i, b_down_j):
    raise NotImplementedError("write your pallas kernel here")


# speedup vs baseline: 1.3830x; 1.3830x over previous
import jax
import jax.numpy as jnp
from jax.experimental import pallas as pl
from jax.experimental.pallas import tpu as pltpu

_F32 = jnp.float32
_BF16 = jnp.bfloat16


def _ru(v, m):
    return ((v + m - 1) // m) * m


def _proj_kernel(h_ref, hu_ref, hd_ref, wia_ref, wju_ref, wjd_ref, bj_ref,
                 ziu_ref, zid_ref, zju_ref, zjd_ref):
    H = ziu_ref.shape[1]
    zi = jnp.dot(h_ref[...], wia_ref[...], preferred_element_type=_F32)
    ziu_ref[...] = zi[:, :H]
    zid_ref[...] = zi[:, H:]
    zju_ref[...] = (jnp.dot(hu_ref[...], wju_ref[...],
                            preferred_element_type=_F32) + bj_ref[:, :H])
    zjd_ref[...] = (jnp.dot(hd_ref[...], wjd_ref[...],
                            preferred_element_type=_F32) + bj_ref[:, H:])


def _edge_kernel(feat_ref, bi_ref, w1x_ref, w2_ref, b2_ref,
                 cw1_ref, cb1_ref, cw2_ref, cb2_ref, out_ref):
    e = pl.program_id(1)
    H = w2_ref.shape[0]
    tE = feat_ref.shape[0]
    n_rows = out_ref.shape[0]

    @pl.when(e == 0)
    def _():
        out_ref[...] = jnp.zeros_like(out_ref)

    feat = feat_ref[...]
    ze = feat[:, :H].astype(_F32)
    xf = feat[:, H:].astype(_F32)

    x_msg = jnp.sum(xf * xf, axis=-1, keepdims=True)
    z1 = ze + x_msg * w1x_ref[...]
    m_e = (jnp.dot(jnp.maximum(z1, 0.0).astype(_BF16), w2_ref[...],
                   preferred_element_type=_F32) + b2_ref[...])

    s1 = jnp.maximum(
        jnp.dot(m_e.astype(_BF16), cw1_ref[...],
                preferred_element_type=_F32) + cb1_ref[...], 0.0)
    gate = jax.nn.sigmoid(
        jnp.dot(s1.astype(_BF16), cw2_ref[...],
                preferred_element_type=_F32) + cb2_ref[...])
    xs = xf * gate

    payload = jnp.concatenate(
        [m_e.astype(_BF16), xs.astype(_BF16)], axis=-1)
    rows = jax.lax.broadcasted_iota(jnp.int32, (n_rows, tE), 0)
    oneh = (rows == bi_ref[...]).astype(_BF16)
    out_ref[...] = out_ref[...] + jnp.dot(
        oneh, payload, preferred_element_type=_F32)


def _cell_kernel(cw_ref, cin_ref, agg_ref, w1_ref, b1_ref, w2_ref, b2_ref,
                 out_ref):
    H = b1_ref.shape[1]
    Fp = w1_ref.shape[0] - 2 * H
    cin = cin_ref[...]
    h_p = cin[:, :Fp]
    x_p = cin[:, Fp:]
    Dp = x_p.shape[1]

    agg = agg_ref[...]
    m_up = agg[:, :H]
    xs_up = agg[:, H:H + Dp]
    m_dn = agg[:, H + Dp:2 * H + Dp]
    xs_dn = agg[:, 2 * H + Dp:]

    lhs = jnp.concatenate([h_p, m_up, m_dn], axis=-1).astype(_BF16)
    z1 = (jnp.dot(lhs, w1_ref[...], preferred_element_type=_F32)
          + b1_ref[...])
    h_upd = (jnp.dot(jnp.maximum(z1, 0.0).astype(_BF16), w2_ref[...],
                     preferred_element_type=_F32) + b2_ref[...])
    h_new = h_p + h_upd
    x_new = x_p + cw_ref[0] * xs_up + cw_ref[1] * xs_dn
    out_ref[...] = jnp.concatenate([h_new, x_new], axis=-1)


def kernel(p_up_W1, p_up_b1, p_up_W2, p_up_b2,
           p_dn_W1, p_dn_b1, p_dn_W2, p_dn_b2,
           p_cu_W1, p_cu_b1, p_cu_W2, p_cu_b2,
           p_cd_W1, p_cd_b1, p_cd_W2, p_cd_b2,
           p_cell_W1, p_cell_b1, p_cell_W2, p_cell_b2, p_cw,
           h, h_up, h_down, x, x_up, x_down,
           b_up_i, b_up_j, b_down_i, b_down_j):
    N, F = h.shape
    Nu, Fu = h_up.shape
    Nd, Fd = h_down.shape
    D = x.shape[1]
    H = p_up_b1.shape[1]
    O = p_cell_b2.shape[1]

    Fp = _ru(F, 128)
    Fup = _ru(Fu, 128)
    Fdp = _ru(Fd, 128)
    Dp = _ru(D, 128)
    Op = _ru(O, 128)
    slab = H + Dp

    tP = min(512, _ru(max(N, Nu, Nd), 8))
    Np = _ru(max(N, Nu, Nd), tP)
    Eu, Ed = int(b_up_i.shape[0]), int(b_down_i.shape[0])
    tE = min(512, _ru(max(Eu, Ed, 1), 8))
    E_pad = _ru(max(Eu, Ed, 1), tE)
    n_e = E_pad // tE
    vmem_lim = 48 * 2**20

    wia = jnp.pad(jnp.concatenate([p_up_W1[:F], p_dn_W1[:F]], axis=1),
                  ((0, Fp - F), (0, 0))).astype(_BF16)
    wju = jnp.pad(p_up_W1[F:F + Fu], ((0, Fup - Fu), (0, 0))).astype(_BF16)
    wjd = jnp.pad(p_dn_W1[F:F + Fd], ((0, Fdp - Fd), (0, 0))).astype(_BF16)
    bj = jnp.concatenate([p_up_b1, p_dn_b1], axis=1)

    w1x_s = jnp.stack([p_up_W1[F + Fu:F + Fu + 1],
                       p_dn_W1[F + Fd:F + Fd + 1]])
    w2_s = jnp.stack([p_up_W2, p_dn_W2]).astype(_BF16)
    b2_s = jnp.stack([p_up_b2, p_dn_b2])
    cw1_s = jnp.stack([p_cu_W1, p_cd_W1]).astype(_BF16)
    cb1_s = jnp.stack([p_cu_b1, p_cd_b1])
    cw2_s = jnp.stack([jnp.tile(p_cu_W2, (1, H)),
                       jnp.tile(p_cd_W2, (1, H))]).astype(_BF16)
    cb2_s = jnp.stack([jnp.tile(p_cu_b2, (1, H)),
                       jnp.tile(p_cd_b2, (1, H))])

    h_b = jnp.pad(h, ((0, Np - N), (0, Fp - F))).astype(_BF16)
    hu_b = jnp.pad(h_up, ((0, Np - Nu), (0, Fup - Fu))).astype(_BF16)
    hd_b = jnp.pad(h_down, ((0, Np - Nd), (0, Fdp - Fd))).astype(_BF16)

    proj = pl.pallas_call(
        _proj_kernel,
        grid=(Np // tP,),
        in_specs=[
            pl.BlockSpec((tP, Fp), lambda i: (i, 0)),
            pl.BlockSpec((tP, Fup), lambda i: (i, 0)),
            pl.BlockSpec((tP, Fdp), lambda i: (i, 0)),
            pl.BlockSpec((Fp, 2 * H), lambda i: (0, 0)),
            pl.BlockSpec((Fup, H), lambda i: (0, 0)),
            pl.BlockSpec((Fdp, H), lambda i: (0, 0)),
            pl.BlockSpec((1, 2 * H), lambda i: (0, 0)),
        ],
        out_specs=[pl.BlockSpec((tP, H), lambda i: (i, 0))] * 4,
        out_shape=[jax.ShapeDtypeStruct((Np, H), _F32)] * 4,
        compiler_params=pltpu.CompilerParams(
            dimension_semantics=("parallel",), vmem_limit_bytes=vmem_lim),
    )
    ziu, zid, zju, zjd = proj(h_b, hu_b, hd_b, wia, wju, wjd, bj)

    bui = b_up_i.astype(jnp.int32)
    buj = b_up_j.astype(jnp.int32)
    bdi = b_down_i.astype(jnp.int32)
    bdj = b_down_j.astype(jnp.int32)

    def mk_feat(zi_t, zj_t, x_other, bi, bj_, E):
        ze = jnp.take(zi_t, bi, axis=0) + jnp.take(zj_t, bj_, axis=0)
        xij = (jnp.take(x, bi, axis=0).astype(_F32)
               - jnp.take(x_other, bj_, axis=0).astype(_F32))
        f = jnp.concatenate(
            [ze.astype(_BF16),
             jnp.pad(xij.astype(_BF16), ((0, 0), (0, Dp - D)))], axis=1)
        return jnp.pad(f, ((0, E_pad - E), (0, 0)))

    feat = jnp.stack([mk_feat(ziu, zju, x_up, bui, buj, Eu),
                      mk_feat(zid, zjd, x_down, bdi, bdj, Ed)])

    def pad_bi(idx, E):
        return jnp.pad(idx, (0, E_pad - E), constant_values=Np)

    bi_all = jnp.stack([pad_bi(bui, Eu), pad_bi(bdi, Ed)]).reshape(2, 1, E_pad)

    edge = pl.pallas_call(
        _edge_kernel,
        grid=(2, n_e),
        in_specs=[
            pl.BlockSpec((None, tE, H + Dp), lambda b, e: (b, e, 0)),
            pl.BlockSpec((None, 1, tE), lambda b, e: (b, 0, e)),
            pl.BlockSpec((None, 1, H), lambda b, e: (b, 0, 0)),
            pl.BlockSpec((None, H, H), lambda b, e: (b, 0, 0)),
            pl.BlockSpec((None, 1, H), lambda b, e: (b, 0, 0)),
            pl.BlockSpec((None, H, H), lambda b, e: (b, 0, 0)),
            pl.BlockSpec((None, 1, H), lambda b, e: (b, 0, 0)),
            pl.BlockSpec((None, H, H), lambda b, e: (b, 0, 0)),
            pl.BlockSpec((None, 1, H), lambda b, e: (b, 0, 0)),
        ],
        out_specs=pl.BlockSpec((Np, slab), lambda b, e: (0, b)),
        out_shape=jax.ShapeDtypeStruct((Np, 2 * slab), _F32),
        compiler_params=pltpu.CompilerParams(
            dimension_semantics=("parallel", "arbitrary"),
            vmem_limit_bytes=vmem_lim),
    )
    agg = edge(feat, bi_all, w1x_s, w2_s, b2_s, cw1_s, cb1_s, cw2_s, cb2_s)

    cin = jnp.concatenate(
        [jnp.pad(h.astype(_F32), ((0, Np - N), (0, Fp - F))),
         jnp.pad(x.astype(_F32), ((0, Np - N), (0, Dp - D)))], axis=-1)

    w1c = jnp.concatenate(
        [jnp.pad(p_cell_W1[:F], ((0, Fp - F), (0, 0))),
         p_cell_W1[F:F + 2 * H]], axis=0).astype(_BF16)
    w2c = jnp.pad(p_cell_W2, ((0, 0), (0, Op - O))).astype(_BF16)
    b2c = jnp.pad(p_cell_b2, ((0, 0), (0, Op - O)))
    cw = p_cw.reshape(-1).astype(_F32)

    tN = min(512, _ru(N, 8))
    cell = pl.pallas_call(
        _cell_kernel,
        grid=(Np // tN,),
        in_specs=[
            pl.BlockSpec(memory_space=pltpu.MemorySpace.SMEM),
            pl.BlockSpec((tN, Fp + Dp), lambda i: (i, 0)),
            pl.BlockSpec((tN, 2 * slab), lambda i: (i, 0)),
            pl.BlockSpec((Fp + 2 * H, H), lambda i: (0, 0)),
            pl.BlockSpec((1, H), lambda i: (0, 0)),
            pl.BlockSpec((H, Op), lambda i: (0, 0)),
            pl.BlockSpec((1, Op), lambda i: (0, 0)),
        ],
        out_specs=pl.BlockSpec((tN, Op + Dp), lambda i: (i, 0)),
        out_shape=jax.ShapeDtypeStruct((Np, Op + Dp), _F32),
        compiler_params=pltpu.CompilerParams(
            dimension_semantics=("parallel",), vmem_limit_bytes=vmem_lim),
    )
    out = cell(cw, cin, agg, w1c, p_cell_b1, w2c, b2c)

    return out[:N, :O], out[:N, Op:Op + D]


# speedup vs baseline: 4.1931x; 3.0319x over previous
import jax
import jax.numpy as jnp
from jax.experimental import pallas as pl
from jax.experimental.pallas import tpu as pltpu

_F32 = jnp.float32
_BF16 = jnp.bfloat16


def _ru(v, m):
    return ((v + m - 1) // m) * m


def _proj_kernel(h_ref, hu_ref, hd_ref, x_ref, xu_ref, xd_ref,
                 wia_ref, wju_ref, wjd_ref, bj_ref,
                 ti_ref, tj_ref):
    H = x_ref.shape[1]
    tP = h_ref.shape[0]
    zi = jnp.dot(h_ref[...], wia_ref[...], preferred_element_type=_F32)
    ti_ref[0, 0:2 * tP:2, :] = zi[:, :H]
    ti_ref[0, 1:2 * tP:2, :] = x_ref[...]
    ti_ref[1, 0:2 * tP:2, :] = zi[:, H:]
    ti_ref[1, 1:2 * tP:2, :] = x_ref[...]
    tj_ref[0, 0:2 * tP:2, :] = (jnp.dot(hu_ref[...], wju_ref[...],
                                        preferred_element_type=_F32)
                                + bj_ref[:, :H])
    tj_ref[0, 1:2 * tP:2, :] = xu_ref[...]
    tj_ref[1, 0:2 * tP:2, :] = (jnp.dot(hd_ref[...], wjd_ref[...],
                                        preferred_element_type=_F32)
                                + bj_ref[:, H:])
    tj_ref[1, 1:2 * tP:2, :] = xd_ref[...]


def _make_edge_kernel(tE):
    S = tE + 1

    def _edge_kernel(bi2_ref, bj2_ref, biv_ref, ti_ref, tj_ref,
                     w1x_ref, w2_ref, b2_ref, cw1_ref, cb1_ref,
                     cw2_ref, cb2_ref, out_ref, tile_i, tile_j):
        e = pl.program_id(1)
        H = w2_ref.shape[0]
        n_rows = out_ref.shape[0]

        @pl.when(e == 0)
        def _():
            out_ref[...] = jnp.zeros_like(out_ref)

        for mi in range(tE):
            i2 = pl.multiple_of(bi2_ref[0, mi], 2)
            tile_i[mi:mi + 2 * S:S, :] = ti_ref[pl.ds(i2, 2), :]
            j2 = pl.multiple_of(bj2_ref[0, mi], 2)
            tile_j[mi:mi + 2 * S:S, :] = tj_ref[pl.ds(j2, 2), :]

        ze = tile_i[pl.ds(0, tE), :] + tile_j[pl.ds(0, tE), :]
        xf = tile_i[pl.ds(S, tE), :] - tile_j[pl.ds(S, tE), :]

        x_msg = jnp.sum(xf * xf, axis=-1, keepdims=True)
        z1 = ze + x_msg * w1x_ref[...]
        m_e = (jnp.dot(jnp.maximum(z1, 0.0).astype(_BF16), w2_ref[...],
                       preferred_element_type=_F32) + b2_ref[...])

        s1 = jnp.maximum(
            jnp.dot(m_e.astype(_BF16), cw1_ref[...],
                    preferred_element_type=_F32) + cb1_ref[...], 0.0)
        gate = jax.nn.sigmoid(
            jnp.dot(s1.astype(_BF16), cw2_ref[...],
                    preferred_element_type=_F32) + cb2_ref[...])
        xs = xf * gate

        payload = jnp.concatenate(
            [m_e.astype(_BF16), xs.astype(_BF16)], axis=-1)
        rows = jax.lax.broadcasted_iota(jnp.int32, (n_rows, tE), 0)
        oneh = (rows == biv_ref[...]).astype(_BF16)
        out_ref[...] = out_ref[...] + jnp.dot(
            oneh, payload, preferred_element_type=_F32)

    return _edge_kernel


def _cell_kernel(cw_ref, cin_ref, agg_ref, w1_ref, b1_ref, w2_ref, b2_ref,
                 out_ref):
    H = b1_ref.shape[1]
    Fp = w1_ref.shape[0] - 2 * H
    cin = cin_ref[...]
    h_p = cin[:, :Fp]
    x_p = cin[:, Fp:]
    Dp = x_p.shape[1]

    agg = agg_ref[...]
    m_up = agg[:, :H]
    xs_up = agg[:, H:H + Dp]
    m_dn = agg[:, H + Dp:2 * H + Dp]
    xs_dn = agg[:, 2 * H + Dp:]

    lhs = jnp.concatenate([h_p, m_up, m_dn], axis=-1).astype(_BF16)
    z1 = (jnp.dot(lhs, w1_ref[...], preferred_element_type=_F32)
          + b1_ref[...])
    h_upd = (jnp.dot(jnp.maximum(z1, 0.0).astype(_BF16), w2_ref[...],
                     preferred_element_type=_F32) + b2_ref[...])
    h_new = h_p + h_upd
    x_new = x_p + cw_ref[0] * xs_up + cw_ref[1] * xs_dn
    out_ref[...] = jnp.concatenate([h_new, x_new], axis=-1)


def kernel(p_up_W1, p_up_b1, p_up_W2, p_up_b2,
           p_dn_W1, p_dn_b1, p_dn_W2, p_dn_b2,
           p_cu_W1, p_cu_b1, p_cu_W2, p_cu_b2,
           p_cd_W1, p_cd_b1, p_cd_W2, p_cd_b2,
           p_cell_W1, p_cell_b1, p_cell_W2, p_cell_b2, p_cw,
           h, h_up, h_down, x, x_up, x_down,
           b_up_i, b_up_j, b_down_i, b_down_j):
    N, F = h.shape
    Nu, Fu = h_up.shape
    Nd, Fd = h_down.shape
    D = x.shape[1]
    H = p_up_b1.shape[1]
    O = p_cell_b2.shape[1]

    Fp = _ru(F, 128)
    Fup = _ru(Fu, 128)
    Fdp = _ru(Fd, 128)
    Dp = _ru(D, 128)
    Op = _ru(O, 128)
    slab = H + Dp

    tP = min(512, _ru(max(N, Nu, Nd), 8))
    Np = _ru(max(N, Nu, Nd), tP)
    Eu, Ed = int(b_up_i.shape[0]), int(b_down_i.shape[0])
    tE = min(512, _ru(max(Eu, Ed, 1), 8))
    E_pad = _ru(max(Eu, Ed, 1), tE)
    n_e = E_pad // tE
    vmem_lim = 48 * 2**20

    wia = jnp.pad(jnp.concatenate([p_up_W1[:F], p_dn_W1[:F]], axis=1),
                  ((0, Fp - F), (0, 0))).astype(_BF16)
    wju = jnp.pad(p_up_W1[F:F + Fu], ((0, Fup - Fu), (0, 0))).astype(_BF16)
    wjd = jnp.pad(p_dn_W1[F:F + Fd], ((0, Fdp - Fd), (0, 0))).astype(_BF16)
    bj = jnp.concatenate([p_up_b1, p_dn_b1], axis=1)

    w1x_s = jnp.stack([p_up_W1[F + Fu:F + Fu + 1],
                       p_dn_W1[F + Fd:F + Fd + 1]])
    w2_s = jnp.stack([p_up_W2, p_dn_W2]).astype(_BF16)
    b2_s = jnp.stack([p_up_b2, p_dn_b2])
    cw1_s = jnp.stack([p_cu_W1, p_cd_W1]).astype(_BF16)
    cb1_s = jnp.stack([p_cu_b1, p_cd_b1])
    cw2_s = jnp.stack([jnp.tile(p_cu_W2, (1, H)),
                       jnp.tile(p_cd_W2, (1, H))]).astype(_BF16)
    cb2_s = jnp.stack([jnp.tile(p_cu_b2, (1, H)),
                       jnp.tile(p_cd_b2, (1, H))])

    h_b = jnp.pad(h, ((0, Np - N), (0, Fp - F))).astype(_BF16)
    hu_b = jnp.pad(h_up, ((0, Np - Nu), (0, Fup - Fu))).astype(_BF16)
    hd_b = jnp.pad(h_down, ((0, Np - Nd), (0, Fdp - Fd))).astype(_BF16)
    x_b = jnp.pad(x.astype(_F32), ((0, Np - N), (0, Dp - D)))
    xu_b = jnp.pad(x_up.astype(_F32), ((0, Np - Nu), (0, Dp - D)))
    xd_b = jnp.pad(x_down.astype(_F32), ((0, Np - Nd), (0, Dp - D)))

    proj = pl.pallas_call(
        _proj_kernel,
        grid=(Np // tP,),
        in_specs=[
            pl.BlockSpec((tP, Fp), lambda i: (i, 0)),
            pl.BlockSpec((tP, Fup), lambda i: (i, 0)),
            pl.BlockSpec((tP, Fdp), lambda i: (i, 0)),
            pl.BlockSpec((tP, Dp), lambda i: (i, 0)),
            pl.BlockSpec((tP, Dp), lambda i: (i, 0)),
            pl.BlockSpec((tP, Dp), lambda i: (i, 0)),
            pl.BlockSpec((Fp, 2 * H), lambda i: (0, 0)),
            pl.BlockSpec((Fup, H), lambda i: (0, 0)),
            pl.BlockSpec((Fdp, H), lambda i: (0, 0)),
            pl.BlockSpec((1, 2 * H), lambda i: (0, 0)),
        ],
        out_specs=[pl.BlockSpec((2, 2 * tP, Dp), lambda i: (0, i, 0))] * 2,
        out_shape=[jax.ShapeDtypeStruct((2, 2 * Np, Dp), _F32)] * 2,
        compiler_params=pltpu.CompilerParams(
            dimension_semantics=("parallel",), vmem_limit_bytes=vmem_lim),
    )
    ti_all, tj_all = proj(h_b, hu_b, hd_b, x_b, xu_b, xd_b, wia, wju, wjd, bj)

    def pad_idx(idx, E, cv):
        return jnp.pad(idx.astype(jnp.int32), (0, E_pad - E),
                       constant_values=cv)

    bi2 = jnp.stack([pad_idx(b_up_i, Eu, 0),
                     pad_idx(b_down_i, Ed, 0)]).reshape(2, 1, E_pad) * 2
    bj2 = jnp.stack([pad_idx(b_up_j, Eu, 0),
                     pad_idx(b_down_j, Ed, 0)]).reshape(2, 1, E_pad) * 2
    biv = jnp.stack([pad_idx(b_up_i, Eu, Np),
                     pad_idx(b_down_i, Ed, Np)]).reshape(2, 1, E_pad)

    edge = pl.pallas_call(
        _make_edge_kernel(tE),
        grid=(2, n_e),
        in_specs=[
            pl.BlockSpec((None, 1, tE), lambda b, e: (b, 0, e),
                         memory_space=pltpu.MemorySpace.SMEM),
            pl.BlockSpec((None, 1, tE), lambda b, e: (b, 0, e),
                         memory_space=pltpu.MemorySpace.SMEM),
            pl.BlockSpec((None, 1, tE), lambda b, e: (b, 0, e)),
            pl.BlockSpec((None, 2 * Np, Dp), lambda b, e: (b, 0, 0)),
            pl.BlockSpec((None, 2 * Np, Dp), lambda b, e: (b, 0, 0)),
            pl.BlockSpec((None, 1, H), lambda b, e: (b, 0, 0)),
            pl.BlockSpec((None, H, H), lambda b, e: (b, 0, 0)),
            pl.BlockSpec((None, 1, H), lambda b, e: (b, 0, 0)),
            pl.BlockSpec((None, H, H), lambda b, e: (b, 0, 0)),
            pl.BlockSpec((None, 1, H), lambda b, e: (b, 0, 0)),
            pl.BlockSpec((None, H, H), lambda b, e: (b, 0, 0)),
            pl.BlockSpec((None, 1, H), lambda b, e: (b, 0, 0)),
        ],
        out_specs=pl.BlockSpec((Np, slab), lambda b, e: (0, b)),
        out_shape=jax.ShapeDtypeStruct((Np, 2 * slab), _F32),
        scratch_shapes=[pltpu.VMEM(((tE + 1) * 2, Dp), _F32)] * 2,
        compiler_params=pltpu.CompilerParams(
            dimension_semantics=("parallel", "arbitrary"),
            vmem_limit_bytes=vmem_lim),
    )
    agg = edge(bi2, bj2, biv, ti_all, tj_all,
               w1x_s, w2_s, b2_s, cw1_s, cb1_s, cw2_s, cb2_s)

    cin = jnp.concatenate(
        [jnp.pad(h.astype(_F32), ((0, Np - N), (0, Fp - F))),
         jnp.pad(x.astype(_F32), ((0, Np - N), (0, Dp - D)))], axis=-1)

    w1c = jnp.concatenate(
        [jnp.pad(p_cell_W1[:F], ((0, Fp - F), (0, 0))),
         p_cell_W1[F:F + 2 * H]], axis=0).astype(_BF16)
    w2c = jnp.pad(p_cell_W2, ((0, 0), (0, Op - O))).astype(_BF16)
    b2c = jnp.pad(p_cell_b2, ((0, 0), (0, Op - O)))
    cw = p_cw.reshape(-1).astype(_F32)

    tN = min(512, _ru(N, 8))
    cell = pl.pallas_call(
        _cell_kernel,
        grid=(Np // tN,),
        in_specs=[
            pl.BlockSpec(memory_space=pltpu.MemorySpace.SMEM),
            pl.BlockSpec((tN, Fp + Dp), lambda i: (i, 0)),
            pl.BlockSpec((tN, 2 * slab), lambda i: (i, 0)),
            pl.BlockSpec((Fp + 2 * H, H), lambda i: (0, 0)),
            pl.BlockSpec((1, H), lambda i: (0, 0)),
            pl.BlockSpec((H, Op), lambda i: (0, 0)),
            pl.BlockSpec((1, Op), lambda i: (0, 0)),
        ],
        out_specs=pl.BlockSpec((tN, Op + Dp), lambda i: (i, 0)),
        out_shape=jax.ShapeDtypeStruct((Np, Op + Dp), _F32),
        compiler_params=pltpu.CompilerParams(
            dimension_semantics=("parallel",), vmem_limit_bytes=vmem_lim),
    )
    out = cell(cw, cin, agg, w1c, p_cell_b1, w2c, b2c)

    return out[:N, :O], out[:N, Op:Op + D]
